# Initial kernel scaffold; baseline (speedup 1.0000x reference)
import jax
import jax.numpy as jnp
from jax.experimental import pallas as pl


def kernel(inputs, targets, features):
    raise NotImplementedError("write your pallas kernel here")


# speedup vs baseline: 3.9907x; 3.9907x over previous
import functools

import jax
import jax.numpy as jnp
from jax.experimental import pallas as pl
from jax.experimental.pallas import tpu as pltpu

TEMP = 0.05
LOGIT_BOUND = 1.0 / TEMP

B = 1024
D = 256
K = 8192
KBLK = 1024
NSTEPS = K // KBLK


def _loss_kernel(x_ref, t_ref, f_ref, out_ref, inv_ref, acc_ref, tgt_ref):
    k = pl.program_id(0)

    @pl.when(k == 0)
    def _init():
        x = x_ref[...]
        norm = jnp.sqrt(jnp.sum(x * x, axis=1, keepdims=True))
        inv_ref[...] = 1.0 / (jnp.maximum(norm, 1e-12) * TEMP)
        acc_ref[...] = jnp.zeros_like(acc_ref)
        tgt_ref[...] = jnp.zeros_like(tgt_ref)

    s = jax.lax.dot_general(
        x_ref[...], f_ref[...],
        dimension_numbers=(((1,), (1,)), ((), ())),
        preferred_element_type=jnp.float32,
    )
    s = s * inv_ref[...]

    acc_ref[...] += jnp.sum(jnp.exp(s - LOGIT_BOUND), axis=1, keepdims=True)

    cols = k * KBLK + jax.lax.broadcasted_iota(jnp.int32, (B, KBLK), 1)
    mask = cols == t_ref[...]
    tgt_ref[...] += jnp.sum(jnp.where(mask, s, 0.0), axis=1, keepdims=True)

    @pl.when(k == NSTEPS - 1)
    def _fini():
        lse = jnp.log(acc_ref[...]) + LOGIT_BOUND
        out_ref[...] = jnp.mean(lse - tgt_ref[...], keepdims=True)


@jax.jit
def _run(inputs, targets, features):
    t2d = targets.astype(jnp.int32).reshape(B, 1)
    out = pl.pallas_call(
        _loss_kernel,
        grid=(NSTEPS,),
        in_specs=[
            pl.BlockSpec((B, D), lambda k: (0, 0)),
            pl.BlockSpec((B, 1), lambda k: (0, 0)),
            pl.BlockSpec((KBLK, D), lambda k: (k, 0)),
        ],
        out_specs=pl.BlockSpec((1, 1), lambda k: (0, 0)),
        out_shape=jax.ShapeDtypeStruct((1, 1), jnp.float32),
        scratch_shapes=[
            pltpu.VMEM((B, 1), jnp.float32),
            pltpu.VMEM((B, 1), jnp.float32),
            pltpu.VMEM((B, 1), jnp.float32),
        ],
    )(inputs, t2d, features)
    return out[0, 0]


def kernel(inputs, targets, features):
    return _run(inputs, targets, features)
